# baseline (speedup 1.0000x reference)
import jax
import jax.numpy as jnp
from jax.experimental import pallas as pl
from jax.experimental.pallas import tpu as pltpu

_THRESH2 = 100.0
_NORM_FACTOR = 100.0
_PB = 128
_L = 128


def _tile_lanes(x, n):
    while x.shape[-1] < n:
        x = jnp.concatenate([x, x], axis=-1)
    return x


def _sum_lane_tiles(x, n):
    while x.shape[-1] > n:
        h = x.shape[-1] // 2
        x = x[:, :h] + x[:, h:]
    return x


def _fused_kernel(hlT_ref, xlT_ref, hpT_ref, xpT_ref, pmT_ref,
                  W1_ref, wdb_ref, W2_ref, aW3_ref,
                  ab2_ref, ab3_ref, vb2_ref, cb2_ref,
                  hout_ref, xout_ref):
    pj = pl.program_id(1)

    hlT = hlT_ref[0]
    xlT = xlT_ref[0]
    hpT = hpT_ref[0]
    xpT = xpT_ref[0]
    pmT = pmT_ref[0]

    L = hlT.shape[1]
    P = hpT.shape[1]
    N = L * P
    f32 = jnp.float32
    bf16 = jnp.bfloat16

    hl_t = _tile_lanes(hlT.astype(bf16), N)
    hp_s = jnp.repeat(hpT.astype(bf16), L, axis=1)
    feat = jnp.concatenate([hl_t, hp_s], axis=0)

    geo = jnp.concatenate([xpT, pmT], axis=0)
    geo_s = jnp.repeat(geo, L, axis=1)
    xl_t = _tile_lanes(xlT, N)
    rel = xl_t - geo_s[0:3]
    d2 = rel[0:1] * rel[0:1] + rel[1:2] * rel[1:2] + rel[2:3] * rel[2:3]
    d2o = jnp.concatenate([d2, jnp.ones((1, N), f32)], axis=0)

    pre = (jnp.dot(W1_ref[...], feat, preferred_element_type=f32)
           + jnp.dot(wdb_ref[...], d2o, preferred_element_type=f32))
    act = jax.nn.silu(pre).astype(bf16)

    out65 = jnp.dot(W2_ref[...], act, preferred_element_type=f32)

    a_h = jax.nn.silu(out65[0:32] + ab2_ref[...]).astype(bf16)
    a = jax.nn.sigmoid(jnp.dot(aW3_ref[...], a_h,
                               preferred_element_type=f32)
                       + ab3_ref[...])
    v = out65[32:64] + vb2_ref[...]
    cw = jnp.tanh(out65[64:65] + cb2_ref[...])

    edge = (d2 < _THRESH2).astype(f32)
    dist = jnp.sqrt(d2 + 1e-8)
    inv = 1.0 / (dist + 1e-8)
    pe = geo_s[3:4] * edge
    s = a * pe
    t = cw * pe * inv

    h_contrib = _sum_lane_tiles(v * s, L)
    x_contrib = _sum_lane_tiles(rel * t, L)

    @pl.when(pj == 0)
    def _init():
        hout_ref[0] = h_contrib
        xout_ref[0] = x_contrib

    @pl.when(pj != 0)
    def _acc():
        hout_ref[0] += h_contrib
        xout_ref[0] += x_contrib


@jax.jit
def kernel(h_ligand, x_ligand, h_protein, x_protein, ligand_mask, protein_mask,
           att_W1, att_b1, att_W2, att_b2, att_W3, att_b3,
           val_W1, val_b1, val_W2, val_b2,
           coord_W1, coord_b1, coord_W2, coord_b2):
    bs, n_lig, lig_nf = h_ligand.shape
    n_prot = h_protein.shape[1]
    prot_nf = h_protein.shape[2]
    hidden = att_W2.shape[0]
    f32 = jnp.float32
    bf16 = jnp.bfloat16

    def w1_row(W1):
        Wl = (W1[:, :lig_nf] if W1.shape[1] == lig_nf + prot_nf + 1
              else jnp.zeros((hidden, lig_nf), f32))
        return jnp.concatenate([Wl, W1[:, -prot_nf - 1:-1]], axis=1)

    W1big = jnp.concatenate(
        [w1_row(att_W1), w1_row(val_W1), w1_row(coord_W1)],
        axis=0).astype(bf16)
    wdb = jnp.concatenate([
        jnp.concatenate([att_W1[:, -1:], att_b1.reshape(hidden, 1)], axis=1),
        jnp.concatenate([val_W1[:, -1:], val_b1.reshape(hidden, 1)], axis=1),
        jnp.concatenate([coord_W1[:, -1:], coord_b1.reshape(hidden, 1)],
                        axis=1)], axis=0)

    z32 = jnp.zeros((hidden, hidden), f32)
    z1 = jnp.zeros((1, hidden), f32)
    W2big = jnp.concatenate([
        jnp.concatenate([att_W2, z32, z32], axis=1),
        jnp.concatenate([z32, val_W2, z32], axis=1),
        jnp.concatenate([z1, z1, coord_W2], axis=1)],
        axis=0).astype(bf16)

    hlT = h_ligand.transpose(0, 2, 1)
    xlT = x_ligand.transpose(0, 2, 1)
    hpT = h_protein.transpose(0, 2, 1)
    xpT = x_protein.transpose(0, 2, 1)
    pmT = protein_mask.transpose(0, 2, 1)

    grid = (bs, n_prot // _PB)

    def full(shape):
        return pl.BlockSpec(shape, lambda b, p: (0,) * len(shape))

    hout, xout = pl.pallas_call(
        _fused_kernel,
        grid=grid,
        in_specs=[
            pl.BlockSpec((1, lig_nf, n_lig), lambda b, p: (b, 0, 0)),
            pl.BlockSpec((1, 3, n_lig), lambda b, p: (b, 0, 0)),
            pl.BlockSpec((1, prot_nf, _PB), lambda b, p: (b, 0, p)),
            pl.BlockSpec((1, 3, _PB), lambda b, p: (b, 0, p)),
            pl.BlockSpec((1, 1, _PB), lambda b, p: (b, 0, p)),
            full((3 * hidden, 2 * hidden)), full((3 * hidden, 2)),
            full((65, 3 * hidden)), full((1, hidden)),
            full((hidden, 1)), full((1, 1)), full((hidden, 1)), full((1, 1)),
        ],
        out_specs=[
            pl.BlockSpec((1, lig_nf, n_lig), lambda b, p: (b, 0, 0)),
            pl.BlockSpec((1, 3, n_lig), lambda b, p: (b, 0, 0)),
        ],
        out_shape=[
            jax.ShapeDtypeStruct((bs, lig_nf, n_lig), f32),
            jax.ShapeDtypeStruct((bs, 3, n_lig), f32),
        ],
        compiler_params=pltpu.CompilerParams(
            dimension_semantics=("parallel", "arbitrary")),
    )(hlT, xlT, hpT, xpT, pmT,
      W1big, wdb, W2big, att_W3.astype(bf16),
      att_b2.reshape(hidden, 1), att_b3.reshape(1, 1),
      val_b2.reshape(lig_nf, 1), coord_b2.reshape(1, 1))

    h_cross = hout.transpose(0, 2, 1) * (ligand_mask / _NORM_FACTOR)
    x_cross = xout.transpose(0, 2, 1) * (ligand_mask / _NORM_FACTOR)
    return (h_cross, x_cross)


# speedup vs baseline: 1.1271x; 1.1271x over previous
import jax
import jax.numpy as jnp
from jax.experimental import pallas as pl
from jax.experimental.pallas import tpu as pltpu

_THRESH2 = 100.0
_NORM_FACTOR = 100.0
_PB = 256
_L = 128


def _tile_lanes(x, n):
    while x.shape[-1] < n:
        x = jnp.concatenate([x, x], axis=-1)
    return x


def _sum_lane_tiles(x, n):
    while x.shape[-1] > n:
        h = x.shape[-1] // 2
        x = x[:, :h] + x[:, h:]
    return x


def _fused_kernel(hlT_ref, xlT_ref, hpT_ref, xpT_ref, pmT_ref,
                  W1_ref, W2_ref, aW3_ref, ab2_ref, ab3_ref, vb2_ref, cb2_ref,
                  hout_ref, xout_ref):
    pj = pl.program_id(1)

    hlT = hlT_ref[0]
    xlT = xlT_ref[0]
    hpT = hpT_ref[0]
    xpT = xpT_ref[0]
    pmT = pmT_ref[0]

    L = hlT.shape[1]
    P = hpT.shape[1]
    N = L * P

    sl = jnp.sum(xlT * xlT, axis=0, keepdims=True)
    sp = jnp.sum(xpT * xpT, axis=0, keepdims=True)
    ones_l = jnp.ones((1, L), jnp.float32)

    l_small = jnp.concatenate([hlT, xlT, sl, ones_l], axis=0)
    s_small = jnp.concatenate([hpT, xpT, sp, pmT], axis=0)

    l_t = _tile_lanes(l_small, N)
    s_s = jnp.repeat(s_small, L, axis=1)

    prod = l_t[32:35] * s_s[32:35]
    feat = jnp.concatenate([l_t, s_s, prod], axis=0)

    out97 = jnp.dot(W1_ref[...], feat, preferred_element_type=jnp.float32)

    d2 = out97[96:97]
    act = jax.nn.silu(out97[0:96])

    out65 = jnp.dot(W2_ref[...], act, preferred_element_type=jnp.float32)

    a_h = jax.nn.silu(out65[0:32] + ab2_ref[...])
    a = jax.nn.sigmoid(jnp.dot(aW3_ref[...], a_h,
                               preferred_element_type=jnp.float32)
                       + ab3_ref[...])
    v = out65[32:64] + vb2_ref[...]
    cw = jnp.tanh(out65[64:65] + cb2_ref[...])

    edge = (d2 < _THRESH2).astype(jnp.float32)
    dist = jnp.sqrt(d2 + 1e-8)
    inv = 1.0 / (dist + 1e-8)
    pe = s_s[36:37] * edge
    s = a * pe
    t = cw * pe * inv

    h_contrib = _sum_lane_tiles(v * s, L)

    rel = l_t[32:35] - s_s[32:35]
    x_contrib = _sum_lane_tiles(rel * t, L)

    @pl.when(pj == 0)
    def _init():
        hout_ref[0] = h_contrib
        xout_ref[0] = x_contrib

    @pl.when(pj != 0)
    def _acc():
        hout_ref[0] += h_contrib
        xout_ref[0] += x_contrib


@jax.jit
def kernel(h_ligand, x_ligand, h_protein, x_protein, ligand_mask, protein_mask,
           att_W1, att_b1, att_W2, att_b2, att_W3, att_b3,
           val_W1, val_b1, val_W2, val_b2,
           coord_W1, coord_b1, coord_W2, coord_b2):
    bs, n_lig, lig_nf = h_ligand.shape
    n_prot = h_protein.shape[1]
    prot_nf = h_protein.shape[2]
    hidden = att_W2.shape[0]
    f32 = jnp.float32

    def w1_rows(W1, b1):
        Wl = (W1[:, :lig_nf] if W1.shape[1] == lig_nf + prot_nf + 1
              else jnp.zeros((hidden, lig_nf), f32))
        Wp = W1[:, -prot_nf - 1:-1]
        wd = W1[:, -1:]
        z3 = jnp.zeros((hidden, 3), f32)
        zc = jnp.zeros((hidden, 1), f32)
        return jnp.concatenate(
            [Wl, z3, wd, b1.reshape(hidden, 1),
             Wp, z3, wd, zc,
             jnp.broadcast_to(-2.0 * wd, (hidden, 3))], axis=1)

    d2_row = jnp.zeros((1, 77), f32).at[0, 35].set(1.0).at[0, 72].set(1.0) \
        .at[0, 74:77].set(-2.0)
    W1big = jnp.concatenate([
        w1_rows(att_W1, att_b1),
        w1_rows(val_W1, val_b1),
        w1_rows(coord_W1, coord_b1),
        d2_row], axis=0)

    z32 = jnp.zeros((hidden, hidden), f32)
    z1 = jnp.zeros((1, hidden), f32)
    W2big = jnp.concatenate([
        jnp.concatenate([att_W2, z32, z32], axis=1),
        jnp.concatenate([z32, val_W2, z32], axis=1),
        jnp.concatenate([z1, z1, coord_W2], axis=1)], axis=0)

    hlT = h_ligand.transpose(0, 2, 1)
    xlT = x_ligand.transpose(0, 2, 1)
    hpT = h_protein.transpose(0, 2, 1)
    xpT = x_protein.transpose(0, 2, 1)
    pmT = protein_mask.transpose(0, 2, 1)

    grid = (bs, n_prot // _PB)

    def full(shape):
        return pl.BlockSpec(shape, lambda b, p: (0,) * len(shape))

    hout, xout = pl.pallas_call(
        _fused_kernel,
        grid=grid,
        in_specs=[
            pl.BlockSpec((1, lig_nf, n_lig), lambda b, p: (b, 0, 0)),
            pl.BlockSpec((1, 3, n_lig), lambda b, p: (b, 0, 0)),
            pl.BlockSpec((1, prot_nf, _PB), lambda b, p: (b, 0, p)),
            pl.BlockSpec((1, 3, _PB), lambda b, p: (b, 0, p)),
            pl.BlockSpec((1, 1, _PB), lambda b, p: (b, 0, p)),
            full((97, 77)), full((65, 96)), full((1, hidden)),
            full((hidden, 1)), full((1, 1)), full((hidden, 1)), full((1, 1)),
        ],
        out_specs=[
            pl.BlockSpec((1, lig_nf, n_lig), lambda b, p: (b, 0, 0)),
            pl.BlockSpec((1, 3, n_lig), lambda b, p: (b, 0, 0)),
        ],
        out_shape=[
            jax.ShapeDtypeStruct((bs, lig_nf, n_lig), f32),
            jax.ShapeDtypeStruct((bs, 3, n_lig), f32),
        ],
        compiler_params=pltpu.CompilerParams(
            dimension_semantics=("parallel", "arbitrary")),
    )(hlT, xlT, hpT, xpT, pmT,
      W1big, W2big, att_W3,
      att_b2.reshape(hidden, 1), att_b3.reshape(1, 1),
      val_b2.reshape(lig_nf, 1), coord_b2.reshape(1, 1))

    h_cross = hout.transpose(0, 2, 1) * (ligand_mask / _NORM_FACTOR)
    x_cross = xout.transpose(0, 2, 1) * (ligand_mask / _NORM_FACTOR)
    return (h_cross, x_cross)


# speedup vs baseline: 1.1567x; 1.0263x over previous
import jax
import jax.numpy as jnp
from jax.experimental import pallas as pl
from jax.experimental.pallas import tpu as pltpu

_THRESH2 = 100.0
_NORM_FACTOR = 100.0
_PB = 256
_L = 128


def _tile_lanes(x, n):
    while x.shape[-1] < n:
        x = jnp.concatenate([x, x], axis=-1)
    return x


def _sum_lane_tiles(x, n):
    while x.shape[-1] > n:
        h = x.shape[-1] // 2
        x = x[:, :h] + x[:, h:]
    return x


def _fused_kernel(hlT_ref, xlT_ref, hpT_ref, xpT_ref, pmT_ref,
                  W1_ref, W2_ref, aW3_ref, ab2_ref, ab3_ref, vb2_ref, cb2_ref,
                  hout_ref, xout_ref):
    pj = pl.program_id(1)

    hlT = hlT_ref[0]
    xlT = xlT_ref[0]
    hpT = hpT_ref[0]
    xpT = xpT_ref[0]
    pmT = pmT_ref[0]

    L = hlT.shape[1]
    P = hpT.shape[1]
    N = L * P

    sl = jnp.sum(xlT * xlT, axis=0, keepdims=True)
    sp = jnp.sum(xpT * xpT, axis=0, keepdims=True)
    ones_l = jnp.ones((1, L), jnp.float32)

    l_small = jnp.concatenate([hlT, xlT, sl, ones_l], axis=0)
    s_small = jnp.concatenate([hpT, xpT, sp, pmT], axis=0)

    bf16 = jnp.bfloat16
    l_t = _tile_lanes(l_small.astype(bf16), N)
    s_s = jnp.repeat(s_small.astype(bf16), L, axis=1)

    prod = l_t[32:35] * s_s[32:35]
    feat = jnp.concatenate([l_t, s_s, prod], axis=0)

    out96 = jnp.dot(W1_ref[...], feat, preferred_element_type=jnp.float32)
    act = jax.nn.silu(out96)

    xl_t = _tile_lanes(xlT, N)
    geo_s = jnp.repeat(jnp.concatenate([xpT, pmT], axis=0), L, axis=1)
    rel = xl_t - geo_s[0:3]
    d2 = rel[0:1] * rel[0:1] + rel[1:2] * rel[1:2] + rel[2:3] * rel[2:3]

    out65 = jnp.dot(W2_ref[...], act, preferred_element_type=jnp.float32)

    a_h = jax.nn.silu(out65[0:32] + ab2_ref[...])
    a = jax.nn.sigmoid(jnp.dot(aW3_ref[...], a_h,
                               preferred_element_type=jnp.float32)
                       + ab3_ref[...])
    v = out65[32:64] + vb2_ref[...]
    cw = jnp.tanh(out65[64:65] + cb2_ref[...])

    edge = (d2 < _THRESH2).astype(jnp.float32)
    dist = jnp.sqrt(d2 + 1e-8)
    inv = 1.0 / (dist + 1e-8)
    pe = geo_s[3:4] * edge
    s = a * pe
    t = cw * pe * inv

    h_contrib = _sum_lane_tiles(v * s, L)
    x_contrib = _sum_lane_tiles(rel * t, L)

    @pl.when(pj == 0)
    def _init():
        hout_ref[0] = h_contrib
        xout_ref[0] = x_contrib

    @pl.when(pj != 0)
    def _acc():
        hout_ref[0] += h_contrib
        xout_ref[0] += x_contrib


@jax.jit
def kernel(h_ligand, x_ligand, h_protein, x_protein, ligand_mask, protein_mask,
           att_W1, att_b1, att_W2, att_b2, att_W3, att_b3,
           val_W1, val_b1, val_W2, val_b2,
           coord_W1, coord_b1, coord_W2, coord_b2):
    bs, n_lig, lig_nf = h_ligand.shape
    n_prot = h_protein.shape[1]
    prot_nf = h_protein.shape[2]
    hidden = att_W2.shape[0]
    f32 = jnp.float32

    def w1_rows(W1, b1):
        Wl = (W1[:, :lig_nf] if W1.shape[1] == lig_nf + prot_nf + 1
              else jnp.zeros((hidden, lig_nf), f32))
        Wp = W1[:, -prot_nf - 1:-1]
        wd = W1[:, -1:]
        z3 = jnp.zeros((hidden, 3), f32)
        zc = jnp.zeros((hidden, 1), f32)
        return jnp.concatenate(
            [Wl, z3, wd, b1.reshape(hidden, 1),
             Wp, z3, wd, zc,
             jnp.broadcast_to(-2.0 * wd, (hidden, 3))], axis=1)

    W1big = jnp.concatenate([
        w1_rows(att_W1, att_b1),
        w1_rows(val_W1, val_b1),
        w1_rows(coord_W1, coord_b1)], axis=0).astype(jnp.bfloat16)

    z32 = jnp.zeros((hidden, hidden), f32)
    z1 = jnp.zeros((1, hidden), f32)
    W2big = jnp.concatenate([
        jnp.concatenate([att_W2, z32, z32], axis=1),
        jnp.concatenate([z32, val_W2, z32], axis=1),
        jnp.concatenate([z1, z1, coord_W2], axis=1)], axis=0)

    hlT = h_ligand.transpose(0, 2, 1)
    xlT = x_ligand.transpose(0, 2, 1)
    hpT = h_protein.transpose(0, 2, 1)
    xpT = x_protein.transpose(0, 2, 1)
    pmT = protein_mask.transpose(0, 2, 1)

    grid = (bs, n_prot // _PB)

    def full(shape):
        return pl.BlockSpec(shape, lambda b, p: (0,) * len(shape))

    hout, xout = pl.pallas_call(
        _fused_kernel,
        grid=grid,
        in_specs=[
            pl.BlockSpec((1, lig_nf, n_lig), lambda b, p: (b, 0, 0)),
            pl.BlockSpec((1, 3, n_lig), lambda b, p: (b, 0, 0)),
            pl.BlockSpec((1, prot_nf, _PB), lambda b, p: (b, 0, p)),
            pl.BlockSpec((1, 3, _PB), lambda b, p: (b, 0, p)),
            pl.BlockSpec((1, 1, _PB), lambda b, p: (b, 0, p)),
            full((96, 77)), full((65, 96)), full((1, hidden)),
            full((hidden, 1)), full((1, 1)), full((hidden, 1)), full((1, 1)),
        ],
        out_specs=[
            pl.BlockSpec((1, lig_nf, n_lig), lambda b, p: (b, 0, 0)),
            pl.BlockSpec((1, 3, n_lig), lambda b, p: (b, 0, 0)),
        ],
        out_shape=[
            jax.ShapeDtypeStruct((bs, lig_nf, n_lig), f32),
            jax.ShapeDtypeStruct((bs, 3, n_lig), f32),
        ],
        compiler_params=pltpu.CompilerParams(
            dimension_semantics=("parallel", "arbitrary")),
    )(hlT, xlT, hpT, xpT, pmT,
      W1big, W2big, att_W3,
      att_b2.reshape(hidden, 1), att_b3.reshape(1, 1),
      val_b2.reshape(lig_nf, 1), coord_b2.reshape(1, 1))

    h_cross = hout.transpose(0, 2, 1) * (ligand_mask / _NORM_FACTOR)
    x_cross = xout.transpose(0, 2, 1) * (ligand_mask / _NORM_FACTOR)
    return (h_cross, x_cross)
